# baseline (speedup 1.0000x reference)
import functools

import jax
import jax.numpy as jnp
from jax import lax
from jax.experimental import pallas as pl
from jax.experimental.pallas import tpu as pltpu
from jax.experimental.pallas import tpu_sc as plsc

VOCAB = 1000000
EMB = 32
T = 100
U1 = 32
U2 = 16
NCLS = 404
B = 1024


def _sc_gather(emb, idx_flat):
    info = plsc.get_sparse_core_info()
    ncores, nsub = info.num_cores, info.num_subcores
    nw = ncores * nsub
    n = idx_flat.shape[0]
    per_w = n // nw

    mesh = plsc.VectorSubcoreMesh(core_axis_name="c", subcore_axis_name="s")

    @functools.partial(
        pl.kernel,
        mesh=mesh,
        out_type=jax.ShapeDtypeStruct((n, EMB), jnp.float32),
        scratch_types=[
            pltpu.VMEM((per_w,), jnp.int32),
            pltpu.VMEM((per_w, EMB), jnp.float32),
            pltpu.SemaphoreType.DMA,
        ],
        compiler_params=pltpu.CompilerParams(use_tc_tiling_on_sc=False),
    )
    def k(table_hbm, idx_hbm, out_hbm, idx_v, rows_v, sem):
        wid = lax.axis_index("s") * ncores + lax.axis_index("c")
        base = wid * per_w
        pltpu.sync_copy(idx_hbm.at[pl.ds(base, per_w)], idx_v)
        pltpu.async_copy(table_hbm.at[idx_v], rows_v, sem).wait()
        pltpu.sync_copy(rows_v, out_hbm.at[pl.ds(base, per_w)])

    return k(emb, idx_flat)


_RELAYOUT_BLK = 65536
_RELAYOUT_NBLK = -(-VOCAB // _RELAYOUT_BLK)
_RELAYOUT_Q = _RELAYOUT_BLK // 4


def _relayout_body(embt_ref, out_ref):
    blk = embt_ref[:]
    q = _RELAYOUT_Q
    s128 = jnp.concatenate([blk[:, k * q:(k + 1) * q] for k in range(4)],
                           axis=0)
    out_ref[:] = jnp.transpose(s128)


def _relayout_emb(embt):
    return pl.pallas_call(
        _relayout_body,
        grid=(_RELAYOUT_NBLK,),
        in_specs=[pl.BlockSpec((EMB, _RELAYOUT_BLK), lambda i: (0, i))],
        out_specs=pl.BlockSpec((_RELAYOUT_BLK // 4, 128), lambda i: (i, 0)),
        out_shape=jax.ShapeDtypeStruct(
            (_RELAYOUT_NBLK * _RELAYOUT_BLK // 4, 128), jnp.float32),
    )(embt)


def _mm(a, b):
    return lax.dot_general(a, b, (((1,), (0,)), ((), ())),
                           preferred_element_type=jnp.float32)


def _tc_body(x2_ref, w1_ref, b1_ref, w2_ref, b2_ref, wd_ref, bd_ref,
             wc_ref, bc_ref, out_ref, xt_ref, x1_ref):
    def tr_step(t, _):
        blk = x2_ref[pl.ds(pl.multiple_of(t * 256, 256), 256), :]
        y = jnp.transpose(blk)
        base = pl.multiple_of(t * EMB, EMB)
        xt_ref[pl.ds(base, EMB), 0:256] = y[0:32]
        xt_ref[pl.ds(base, EMB), 256:512] = y[32:64]
        xt_ref[pl.ds(base, EMB), 512:768] = y[64:96]
        xt_ref[pl.ds(base, EMB), 768:1024] = y[96:128]
        return 0

    lax.fori_loop(0, T, tr_step, 0)

    w1 = w1_ref[:]
    b1 = b1_ref[:]

    def l1_step(t, carry):
        h, c = carry
        xf = xt_ref[pl.ds(pl.multiple_of(t * EMB, EMB), EMB), :]
        xb = xt_ref[pl.ds(pl.multiple_of((T - 1 - t) * EMB, EMB), EMB), :]
        s = jnp.concatenate([xf, h[0:U1], xb, h[U1:2 * U1]], axis=0)
        z = _mm(w1, s) + b1
        zs = jnp.tanh(z[0:6 * U1]) * 0.5 + 0.5
        g = jnp.tanh(z[6 * U1:8 * U1])
        i = zs[0:2 * U1]
        f = zs[2 * U1:4 * U1]
        o = zs[4 * U1:6 * U1]
        c2 = f * c + i * g
        h2 = o * jnp.tanh(c2)
        x1_ref[pl.ds(pl.multiple_of(t * 2 * U1, 2 * U1), U1), :] = h2[0:U1]
        x1_ref[pl.ds(pl.multiple_of((T - 1 - t) * 2 * U1 + U1, U1), U1), :] = \
            h2[U1:2 * U1]
        return h2, c2

    zero1 = jnp.zeros((2 * U1, B), jnp.float32)
    lax.fori_loop(0, T, l1_step, (zero1, zero1))

    w2 = w2_ref[:]
    b2 = b2_ref[:]

    def l2_step(t, carry):
        h, c = carry
        x1f = x1_ref[pl.ds(pl.multiple_of(t * 2 * U1, 2 * U1), 2 * U1), :]
        x1b = x1_ref[pl.ds(pl.multiple_of((T - 1 - t) * 2 * U1, 2 * U1),
                           2 * U1), :]
        s = jnp.concatenate([x1f, h[0:U2], x1b, h[U2:2 * U2]], axis=0)
        z = _mm(w2, s) + b2
        zs = jnp.tanh(z[0:6 * U2]) * 0.5 + 0.5
        g = jnp.tanh(z[6 * U2:8 * U2])
        i = zs[0:2 * U2]
        f = zs[2 * U2:4 * U2]
        o = zs[4 * U2:6 * U2]
        c2 = f * c + i * g
        h2 = o * jnp.tanh(c2)
        return h2, c2

    zero2 = jnp.zeros((2 * U2, B), jnp.float32)
    h2, _ = lax.fori_loop(0, T, l2_step, (zero2, zero2))

    d = jnp.maximum(_mm(wd_ref[:], h2) + bd_ref[:], 0.0)
    logits = _mm(wc_ref[:], d) + bc_ref[:]
    m = jnp.max(logits, axis=0, keepdims=True)
    e = jnp.exp(logits - m)
    out_ref[:] = e / jnp.sum(e, axis=0, keepdims=True)


def _tc_forward(x2d, w1, b1, w2, b2, wd, bd, wc, bc):
    return pl.pallas_call(
        _tc_body,
        out_shape=jax.ShapeDtypeStruct((NCLS, B), jnp.float32),
        scratch_shapes=[pltpu.VMEM((T * EMB, B), jnp.float32),
                        pltpu.VMEM((T * 2 * U1, B), jnp.float32)],
    )(x2d, w1, b1, w2, b2, wd, bd, wc, bc)


def _pack_lstm_weights(wf_x, wf_h, bf, wb_x, wb_h, bb, u, din):
    af = jnp.concatenate([wf_x, wf_h], axis=0).T
    ab = jnp.concatenate([wb_x, wb_h], axis=0).T
    dpu = din + u
    w = jnp.zeros((8 * u, 2 * dpu), jnp.float32)
    bias = []
    for k, r0 in enumerate((0, u, 3 * u, 2 * u)):
        sc = 0.5 if k < 3 else 1.0
        w = w.at[2 * k * u:(2 * k + 1) * u, 0:dpu].set(sc * af[r0:r0 + u])
        w = w.at[(2 * k + 1) * u:(2 * k + 2) * u, dpu:2 * dpu].set(
            sc * ab[r0:r0 + u])
        bias.append(sc * bf[r0:r0 + u])
        bias.append(sc * bb[r0:r0 + u])
    b = jnp.concatenate(bias)[:, None]
    return w, b


def kernel(inputs, emb, w1f_x, w1f_h, b1f, w1b_x, w1b_h, b1b,
           w2f_x, w2f_h, b2f, w2b_x, w2b_h, b2b, Wd, bd, Wc, bc):
    v = inputs.astype(jnp.int32)
    _bm1, _qm1 = _RELAYOUT_BLK - 1, _RELAYOUT_Q - 1
    _lq = _RELAYOUT_Q.bit_length() - 1
    vp = (v & ~_bm1) + ((v & _qm1) << 2) + ((v & _bm1) >> _lq)
    idx = (vp.astype(jnp.float32).T
           .reshape(T, 4, 256).transpose(0, 2, 1).reshape(-1)
           .astype(jnp.int32))
    emb_lin = _relayout_emb(emb.T).reshape(-1, EMB)
    rows = _sc_gather(emb_lin, idx)
    x2d = rows.reshape(T * 256, 128)

    w1, b1 = _pack_lstm_weights(w1f_x, w1f_h, b1f, w1b_x, w1b_h, b1b,
                                U1, EMB)
    w2, b2 = _pack_lstm_weights(w2f_x, w2f_h, b2f, w2b_x, w2b_h, b2b,
                                U2, 2 * U1)
    out_t = _tc_forward(x2d, w1, b1, w2, b2,
                        Wd.T, bd[:, None], Wc.T, bc[:, None])
    return out_t.T


# speedup vs baseline: 1.0434x; 1.0434x over previous
import functools

import jax
import jax.numpy as jnp
from jax import lax
from jax.experimental import pallas as pl
from jax.experimental.pallas import tpu as pltpu
from jax.experimental.pallas import tpu_sc as plsc

VOCAB = 1000000
EMB = 32
T = 100
U1 = 32
U2 = 16
NCLS = 404
B = 1024


def _sc_gather(emb, idx_flat):
    info = plsc.get_sparse_core_info()
    ncores, nsub = info.num_cores, info.num_subcores
    nw = ncores * nsub
    n = idx_flat.shape[0]
    per_w = n // nw

    mesh = plsc.VectorSubcoreMesh(core_axis_name="c", subcore_axis_name="s")

    @functools.partial(
        pl.kernel,
        mesh=mesh,
        out_type=jax.ShapeDtypeStruct((n, EMB), jnp.float32),
        scratch_types=[
            pltpu.VMEM((per_w,), jnp.int32),
            pltpu.VMEM((per_w, EMB), jnp.float32),
            pltpu.SemaphoreType.DMA,
        ],
        compiler_params=pltpu.CompilerParams(use_tc_tiling_on_sc=False),
    )
    def k(table_hbm, idx_hbm, out_hbm, idx_v, rows_v, sem):
        wid = lax.axis_index("s") * ncores + lax.axis_index("c")
        base = wid * per_w
        pltpu.sync_copy(idx_hbm.at[pl.ds(base, per_w)], idx_v)
        pltpu.async_copy(table_hbm.at[idx_v], rows_v, sem).wait()
        pltpu.sync_copy(rows_v, out_hbm.at[pl.ds(base, per_w)])

    return k(emb, idx_flat)


_RELAYOUT_BLK = 32768
_RELAYOUT_NBLK = -(-VOCAB // _RELAYOUT_BLK)
_RELAYOUT_Q = _RELAYOUT_BLK // 4


def _relayout_body(embt_ref, out_ref):
    blk = embt_ref[:]
    q = _RELAYOUT_Q
    s128 = jnp.concatenate([blk[:, k * q:(k + 1) * q] for k in range(4)],
                           axis=0)
    out_ref[:] = jnp.transpose(s128)


def _relayout_emb(embt):
    return pl.pallas_call(
        _relayout_body,
        grid=(_RELAYOUT_NBLK,),
        in_specs=[pl.BlockSpec((EMB, _RELAYOUT_BLK), lambda i: (0, i))],
        out_specs=pl.BlockSpec((_RELAYOUT_BLK // 4, 128), lambda i: (i, 0)),
        out_shape=jax.ShapeDtypeStruct(
            (_RELAYOUT_NBLK * _RELAYOUT_BLK // 4, 128), jnp.float32),
    )(embt)


def _mm(a, b):
    return lax.dot_general(a, b, (((1,), (0,)), ((), ())),
                           preferred_element_type=jnp.float32)


def _tc_body(x2_ref, w1_ref, b1_ref, w2_ref, b2_ref, wd_ref, bd_ref,
             wc_ref, bc_ref, out_ref, xt_ref, x1_ref):
    def tr_step(t, _):
        blk = x2_ref[pl.ds(pl.multiple_of(t * 256, 256), 256), :]
        y = jnp.transpose(blk)
        base = pl.multiple_of(t * EMB, EMB)
        xt_ref[pl.ds(base, EMB), 0:256] = y[0:32]
        xt_ref[pl.ds(base, EMB), 256:512] = y[32:64]
        xt_ref[pl.ds(base, EMB), 512:768] = y[64:96]
        xt_ref[pl.ds(base, EMB), 768:1024] = y[96:128]
        return 0

    lax.fori_loop(0, T, tr_step, 0)

    w1 = w1_ref[:]
    b1 = b1_ref[:]

    def l1_step(t, carry):
        h, c = carry
        xf = xt_ref[pl.ds(pl.multiple_of(t * EMB, EMB), EMB), :]
        xb = xt_ref[pl.ds(pl.multiple_of((T - 1 - t) * EMB, EMB), EMB), :]
        s = jnp.concatenate([xf, h[0:U1], xb, h[U1:2 * U1]], axis=0)
        z = _mm(w1, s) + b1
        zs = jnp.tanh(z[0:6 * U1]) * 0.5 + 0.5
        g = jnp.tanh(z[6 * U1:8 * U1])
        i = zs[0:2 * U1]
        f = zs[2 * U1:4 * U1]
        o = zs[4 * U1:6 * U1]
        c2 = f * c + i * g
        h2 = o * jnp.tanh(c2)
        x1_ref[pl.ds(pl.multiple_of(t * 2 * U1, 2 * U1), U1), :] = h2[0:U1]
        x1_ref[pl.ds(pl.multiple_of((T - 1 - t) * 2 * U1 + U1, U1), U1), :] = \
            h2[U1:2 * U1]
        return h2, c2

    zero1 = jnp.zeros((2 * U1, B), jnp.float32)
    lax.fori_loop(0, T, l1_step, (zero1, zero1), unroll=2)

    w2 = w2_ref[:]
    b2 = b2_ref[:]

    def l2_step(t, carry):
        h, c = carry
        x1f = x1_ref[pl.ds(pl.multiple_of(t * 2 * U1, 2 * U1), 2 * U1), :]
        x1b = x1_ref[pl.ds(pl.multiple_of((T - 1 - t) * 2 * U1, 2 * U1),
                           2 * U1), :]
        s = jnp.concatenate([x1f, h[0:U2], x1b, h[U2:2 * U2]], axis=0)
        z = _mm(w2, s) + b2
        zs = jnp.tanh(z[0:6 * U2]) * 0.5 + 0.5
        g = jnp.tanh(z[6 * U2:8 * U2])
        i = zs[0:2 * U2]
        f = zs[2 * U2:4 * U2]
        o = zs[4 * U2:6 * U2]
        c2 = f * c + i * g
        h2 = o * jnp.tanh(c2)
        return h2, c2

    zero2 = jnp.zeros((2 * U2, B), jnp.float32)
    h2, _ = lax.fori_loop(0, T, l2_step, (zero2, zero2), unroll=2)

    d = jnp.maximum(_mm(wd_ref[:], h2) + bd_ref[:], 0.0)
    logits = _mm(wc_ref[:], d) + bc_ref[:]
    m = jnp.max(logits, axis=0, keepdims=True)
    e = jnp.exp(logits - m)
    out_ref[:] = e / jnp.sum(e, axis=0, keepdims=True)


def _tc_forward(x2d, w1, b1, w2, b2, wd, bd, wc, bc):
    return pl.pallas_call(
        _tc_body,
        out_shape=jax.ShapeDtypeStruct((NCLS, B), jnp.float32),
        scratch_shapes=[pltpu.VMEM((T * EMB, B), jnp.float32),
                        pltpu.VMEM((T * 2 * U1, B), jnp.float32)],
    )(x2d, w1, b1, w2, b2, wd, bd, wc, bc)


def _pack_lstm_weights(wf_x, wf_h, bf, wb_x, wb_h, bb, u, din):
    af = jnp.concatenate([wf_x, wf_h], axis=0).T
    ab = jnp.concatenate([wb_x, wb_h], axis=0).T
    dpu = din + u
    w = jnp.zeros((8 * u, 2 * dpu), jnp.float32)
    bias = []
    for k, r0 in enumerate((0, u, 3 * u, 2 * u)):
        sc = 0.5 if k < 3 else 1.0
        w = w.at[2 * k * u:(2 * k + 1) * u, 0:dpu].set(sc * af[r0:r0 + u])
        w = w.at[(2 * k + 1) * u:(2 * k + 2) * u, dpu:2 * dpu].set(
            sc * ab[r0:r0 + u])
        bias.append(sc * bf[r0:r0 + u])
        bias.append(sc * bb[r0:r0 + u])
    b = jnp.concatenate(bias)[:, None]
    return w, b


def kernel(inputs, emb, w1f_x, w1f_h, b1f, w1b_x, w1b_h, b1b,
           w2f_x, w2f_h, b2f, w2b_x, w2b_h, b2b, Wd, bd, Wc, bc):
    v = inputs.astype(jnp.int32)
    _bm1, _qm1 = _RELAYOUT_BLK - 1, _RELAYOUT_Q - 1
    _lq = _RELAYOUT_Q.bit_length() - 1
    vp = (v & ~_bm1) + ((v & _qm1) << 2) + ((v & _bm1) >> _lq)
    idx = (vp.astype(jnp.float32).T
           .reshape(T, 4, 256).transpose(0, 2, 1).reshape(-1)
           .astype(jnp.int32))
    emb_lin = _relayout_emb(emb.T).reshape(-1, EMB)
    rows = _sc_gather(emb_lin, idx)
    x2d = rows.reshape(T * 256, 128)

    w1, b1 = _pack_lstm_weights(w1f_x, w1f_h, b1f, w1b_x, w1b_h, b1b,
                                U1, EMB)
    w2, b2 = _pack_lstm_weights(w2f_x, w2f_h, b2f, w2b_x, w2b_h, b2b,
                                U2, 2 * U1)
    out_t = _tc_forward(x2d, w1, b1, w2, b2,
                        Wd.T, bd[:, None], Wc.T, bc[:, None])
    return out_t.T


# speedup vs baseline: 1.0740x; 1.0293x over previous
import functools

import jax
import jax.numpy as jnp
from jax import lax
from jax.experimental import pallas as pl
from jax.experimental.pallas import tpu as pltpu
from jax.experimental.pallas import tpu_sc as plsc

VOCAB = 1000000
EMB = 32
T = 100
U1 = 32
U2 = 16
NCLS = 404
B = 1024


def _sc_gather(emb, idx_flat):
    info = plsc.get_sparse_core_info()
    ncores, nsub = info.num_cores, info.num_subcores
    nw = ncores * nsub
    n = idx_flat.shape[0]
    per_w = n // nw

    mesh = plsc.VectorSubcoreMesh(core_axis_name="c", subcore_axis_name="s")

    @functools.partial(
        pl.kernel,
        mesh=mesh,
        out_type=jax.ShapeDtypeStruct((n, EMB), jnp.float32),
        scratch_types=[
            pltpu.VMEM((per_w,), jnp.int32),
            pltpu.VMEM((per_w, EMB), jnp.float32),
            pltpu.SemaphoreType.DMA,
        ],
        compiler_params=pltpu.CompilerParams(use_tc_tiling_on_sc=False),
    )
    def k(table_hbm, idx_hbm, out_hbm, idx_v, rows_v, sem):
        wid = lax.axis_index("s") * ncores + lax.axis_index("c")
        base = wid * per_w
        pltpu.sync_copy(idx_hbm.at[pl.ds(base, per_w)], idx_v)
        pltpu.async_copy(table_hbm.at[idx_v], rows_v, sem).wait()
        pltpu.sync_copy(rows_v, out_hbm.at[pl.ds(base, per_w)])

    return k(emb, idx_flat)


_RELAYOUT_BLK = 32768
_RELAYOUT_NBLK = -(-VOCAB // _RELAYOUT_BLK)
_RELAYOUT_Q = _RELAYOUT_BLK // 4


def _relayout_body(embt_ref, out_ref):
    blk = embt_ref[:]
    q = _RELAYOUT_Q
    s128 = jnp.concatenate([blk[:, k * q:(k + 1) * q] for k in range(4)],
                           axis=0)
    out_ref[:] = jnp.transpose(s128)


def _relayout_emb(embt):
    return pl.pallas_call(
        _relayout_body,
        grid=(_RELAYOUT_NBLK,),
        in_specs=[pl.BlockSpec((EMB, _RELAYOUT_BLK), lambda i: (0, i))],
        out_specs=pl.BlockSpec((_RELAYOUT_BLK // 4, 128), lambda i: (i, 0)),
        out_shape=jax.ShapeDtypeStruct(
            (_RELAYOUT_NBLK * _RELAYOUT_BLK // 4, 128), jnp.float32),
    )(embt)


def _mm(a, b):
    return lax.dot_general(a, b, (((1,), (0,)), ((), ())),
                           preferred_element_type=jnp.float32)


def _tc_body(x2_ref, w1_ref, b1_ref, w2_ref, b2_ref, wd_ref, bd_ref,
             wc_ref, bc_ref, out_ref, xt_ref, x1_ref):
    def tr_step(t, _):
        blk = x2_ref[pl.ds(pl.multiple_of(t * 256, 256), 256), :]
        y = jnp.transpose(blk)
        base = pl.multiple_of(t * EMB, EMB)
        xt_ref[pl.ds(base, EMB), 0:256] = y[0:32]
        xt_ref[pl.ds(base, EMB), 256:512] = y[32:64]
        xt_ref[pl.ds(base, EMB), 512:768] = y[64:96]
        xt_ref[pl.ds(base, EMB), 768:1024] = y[96:128]
        return 0

    lax.fori_loop(0, T, tr_step, 0)

    w1 = w1_ref[:]
    b1 = b1_ref[:]

    def l1_step(t, carry):
        h, c = carry
        xf = xt_ref[pl.ds(pl.multiple_of(t * EMB, EMB), EMB), :]
        xb = xt_ref[pl.ds(pl.multiple_of((T - 1 - t) * EMB, EMB), EMB), :]
        s = jnp.concatenate([xf, h[0:U1], xb, h[U1:2 * U1]], axis=0)
        z = _mm(w1, s) + b1
        zs = jnp.tanh(z[0:6 * U1]) * 0.5 + 0.5
        g = jnp.tanh(z[6 * U1:8 * U1])
        i = zs[0:2 * U1]
        f = zs[2 * U1:4 * U1]
        o = zs[4 * U1:6 * U1]
        c2 = f * c + i * g
        h2 = o * jnp.tanh(c2)
        x1_ref[pl.ds(pl.multiple_of(t * 2 * U1, 2 * U1), U1), :] = h2[0:U1]
        x1_ref[pl.ds(pl.multiple_of((T - 1 - t) * 2 * U1 + U1, U1), U1), :] = \
            h2[U1:2 * U1]
        return h2, c2

    zero1 = jnp.zeros((2 * U1, B), jnp.float32)
    lax.fori_loop(0, T, l1_step, (zero1, zero1), unroll=4)

    w2 = w2_ref[:]
    b2 = b2_ref[:]

    def l2_step(t, carry):
        h, c = carry
        x1f = x1_ref[pl.ds(pl.multiple_of(t * 2 * U1, 2 * U1), 2 * U1), :]
        x1b = x1_ref[pl.ds(pl.multiple_of((T - 1 - t) * 2 * U1, 2 * U1),
                           2 * U1), :]
        s = jnp.concatenate([x1f, h[0:U2], x1b, h[U2:2 * U2]], axis=0)
        z = _mm(w2, s) + b2
        zs = jnp.tanh(z[0:6 * U2]) * 0.5 + 0.5
        g = jnp.tanh(z[6 * U2:8 * U2])
        i = zs[0:2 * U2]
        f = zs[2 * U2:4 * U2]
        o = zs[4 * U2:6 * U2]
        c2 = f * c + i * g
        h2 = o * jnp.tanh(c2)
        return h2, c2

    zero2 = jnp.zeros((2 * U2, B), jnp.float32)
    h2, _ = lax.fori_loop(0, T, l2_step, (zero2, zero2), unroll=4)

    d = jnp.maximum(_mm(wd_ref[:], h2) + bd_ref[:], 0.0)
    logits = _mm(wc_ref[:], d) + bc_ref[:]
    m = jnp.max(logits, axis=0, keepdims=True)
    e = jnp.exp(logits - m)
    out_ref[:] = e / jnp.sum(e, axis=0, keepdims=True)


def _tc_forward(x2d, w1, b1, w2, b2, wd, bd, wc, bc):
    return pl.pallas_call(
        _tc_body,
        out_shape=jax.ShapeDtypeStruct((NCLS, B), jnp.float32),
        scratch_shapes=[pltpu.VMEM((T * EMB, B), jnp.float32),
                        pltpu.VMEM((T * 2 * U1, B), jnp.float32)],
    )(x2d, w1, b1, w2, b2, wd, bd, wc, bc)


def _pack_lstm_weights(wf_x, wf_h, bf, wb_x, wb_h, bb, u, din):
    af = jnp.concatenate([wf_x, wf_h], axis=0).T
    ab = jnp.concatenate([wb_x, wb_h], axis=0).T
    dpu = din + u
    w = jnp.zeros((8 * u, 2 * dpu), jnp.float32)
    bias = []
    for k, r0 in enumerate((0, u, 3 * u, 2 * u)):
        sc = 0.5 if k < 3 else 1.0
        w = w.at[2 * k * u:(2 * k + 1) * u, 0:dpu].set(sc * af[r0:r0 + u])
        w = w.at[(2 * k + 1) * u:(2 * k + 2) * u, dpu:2 * dpu].set(
            sc * ab[r0:r0 + u])
        bias.append(sc * bf[r0:r0 + u])
        bias.append(sc * bb[r0:r0 + u])
    b = jnp.concatenate(bias)[:, None]
    return w, b


def kernel(inputs, emb, w1f_x, w1f_h, b1f, w1b_x, w1b_h, b1b,
           w2f_x, w2f_h, b2f, w2b_x, w2b_h, b2b, Wd, bd, Wc, bc):
    v = inputs.astype(jnp.int32)
    _bm1, _qm1 = _RELAYOUT_BLK - 1, _RELAYOUT_Q - 1
    _lq = _RELAYOUT_Q.bit_length() - 1
    vp = (v & ~_bm1) + ((v & _qm1) << 2) + ((v & _bm1) >> _lq)
    idx = (vp.astype(jnp.float32).T
           .reshape(T, 4, 256).transpose(0, 2, 1).reshape(-1)
           .astype(jnp.int32))
    emb_lin = _relayout_emb(emb.T).reshape(-1, EMB)
    rows = _sc_gather(emb_lin, idx)
    x2d = rows.reshape(T * 256, 128)

    w1, b1 = _pack_lstm_weights(w1f_x, w1f_h, b1f, w1b_x, w1b_h, b1b,
                                U1, EMB)
    w2, b2 = _pack_lstm_weights(w2f_x, w2f_h, b2f, w2b_x, w2b_h, b2b,
                                U2, 2 * U1)
    out_t = _tc_forward(x2d, w1, b1, w2, b2,
                        Wd.T, bd[:, None], Wc.T, bc[:, None])
    return out_t.T


# speedup vs baseline: 1.1063x; 1.0301x over previous
import functools

import jax
import jax.numpy as jnp
from jax import lax
from jax.experimental import pallas as pl
from jax.experimental.pallas import tpu as pltpu
from jax.experimental.pallas import tpu_sc as plsc

VOCAB = 1000000
EMB = 32
T = 100
U1 = 32
U2 = 16
NCLS = 404
B = 1024


def _sc_gather(emb, idx_flat):
    info = plsc.get_sparse_core_info()
    ncores, nsub = info.num_cores, info.num_subcores
    nw = ncores * nsub
    n = idx_flat.shape[0]
    per_w = n // nw

    mesh = plsc.VectorSubcoreMesh(core_axis_name="c", subcore_axis_name="s")

    @functools.partial(
        pl.kernel,
        mesh=mesh,
        out_type=jax.ShapeDtypeStruct((n, EMB), jnp.float32),
        scratch_types=[
            pltpu.VMEM((per_w,), jnp.int32),
            pltpu.VMEM((per_w, EMB), jnp.float32),
            pltpu.SemaphoreType.DMA,
        ],
        compiler_params=pltpu.CompilerParams(use_tc_tiling_on_sc=False),
    )
    def k(table_hbm, idx_hbm, out_hbm, idx_v, rows_v, sem):
        wid = lax.axis_index("s") * ncores + lax.axis_index("c")
        base = wid * per_w
        pltpu.sync_copy(idx_hbm.at[pl.ds(base, per_w)], idx_v)
        pltpu.async_copy(table_hbm.at[idx_v], rows_v, sem).wait()
        pltpu.sync_copy(rows_v, out_hbm.at[pl.ds(base, per_w)])

    return k(emb, idx_flat)


_RELAYOUT_BLK = 32768
_RELAYOUT_NBLK = -(-VOCAB // _RELAYOUT_BLK)
_RELAYOUT_Q = _RELAYOUT_BLK // 4


def _relayout_body(embt_ref, out_ref):
    blk = embt_ref[:]
    q = _RELAYOUT_Q
    s128 = jnp.concatenate([blk[:, k * q:(k + 1) * q] for k in range(4)],
                           axis=0)
    out_ref[:] = jnp.transpose(s128)


def _relayout_emb(embt):
    return pl.pallas_call(
        _relayout_body,
        grid=(_RELAYOUT_NBLK,),
        in_specs=[pl.BlockSpec((EMB, _RELAYOUT_BLK), lambda i: (0, i))],
        out_specs=pl.BlockSpec((_RELAYOUT_BLK // 4, 128), lambda i: (i, 0)),
        out_shape=jax.ShapeDtypeStruct(
            (_RELAYOUT_NBLK * _RELAYOUT_BLK // 4, 128), jnp.float32),
    )(embt)


def _mm(a, b):
    return lax.dot_general(a, b, (((1,), (0,)), ((), ())),
                           preferred_element_type=jnp.float32)


def _tc_body(x2_ref, w1_ref, b1_ref, w2_ref, b2_ref, wd_ref, bd_ref,
             wc_ref, bc_ref, out_ref, xt_ref, x1_ref):
    def tr_step(t, _):
        blk = x2_ref[pl.ds(pl.multiple_of(t * 256, 256), 256), :]
        y = jnp.transpose(blk)
        base = pl.multiple_of(t * EMB, EMB)
        xt_ref[pl.ds(base, EMB), 0:256] = y[0:32]
        xt_ref[pl.ds(base, EMB), 256:512] = y[32:64]
        xt_ref[pl.ds(base, EMB), 512:768] = y[64:96]
        xt_ref[pl.ds(base, EMB), 768:1024] = y[96:128]
        return 0

    lax.fori_loop(0, T, tr_step, 0, unroll=4)

    w1 = w1_ref[:]
    b1 = b1_ref[:]

    def l1_step(t, carry):
        h, c = carry
        xf = xt_ref[pl.ds(pl.multiple_of(t * EMB, EMB), EMB), :]
        xb = xt_ref[pl.ds(pl.multiple_of((T - 1 - t) * EMB, EMB), EMB), :]
        s = jnp.concatenate([xf, h[0:U1], xb, h[U1:2 * U1]], axis=0)
        z = _mm(w1, s) + b1
        zs = jnp.tanh(z[0:6 * U1]) * 0.5 + 0.5
        g = jnp.tanh(z[6 * U1:8 * U1])
        i = zs[0:2 * U1]
        f = zs[2 * U1:4 * U1]
        o = zs[4 * U1:6 * U1]
        c2 = f * c + i * g
        h2 = o * jnp.tanh(c2)
        x1_ref[pl.ds(pl.multiple_of(t * 2 * U1, 2 * U1), U1), :] = h2[0:U1]
        x1_ref[pl.ds(pl.multiple_of((T - 1 - t) * 2 * U1 + U1, U1), U1), :] = \
            h2[U1:2 * U1]
        return h2, c2

    zero1 = jnp.zeros((2 * U1, B), jnp.float32)
    lax.fori_loop(0, T, l1_step, (zero1, zero1), unroll=5)

    w2 = w2_ref[:]
    b2 = b2_ref[:]

    def l2_step(t, carry):
        h, c = carry
        x1f = x1_ref[pl.ds(pl.multiple_of(t * 2 * U1, 2 * U1), 2 * U1), :]
        x1b = x1_ref[pl.ds(pl.multiple_of((T - 1 - t) * 2 * U1, 2 * U1),
                           2 * U1), :]
        s = jnp.concatenate([x1f, h[0:U2], x1b, h[U2:2 * U2]], axis=0)
        z = _mm(w2, s) + b2
        zs = jnp.tanh(z[0:6 * U2]) * 0.5 + 0.5
        g = jnp.tanh(z[6 * U2:8 * U2])
        i = zs[0:2 * U2]
        f = zs[2 * U2:4 * U2]
        o = zs[4 * U2:6 * U2]
        c2 = f * c + i * g
        h2 = o * jnp.tanh(c2)
        return h2, c2

    zero2 = jnp.zeros((2 * U2, B), jnp.float32)
    h2, _ = lax.fori_loop(0, T, l2_step, (zero2, zero2), unroll=5)

    d = jnp.maximum(_mm(wd_ref[:], h2) + bd_ref[:], 0.0)
    logits = _mm(wc_ref[:], d) + bc_ref[:]
    m = jnp.max(logits, axis=0, keepdims=True)
    e = jnp.exp(logits - m)
    out_ref[:] = e / jnp.sum(e, axis=0, keepdims=True)


def _tc_forward(x2d, w1, b1, w2, b2, wd, bd, wc, bc):
    return pl.pallas_call(
        _tc_body,
        out_shape=jax.ShapeDtypeStruct((NCLS, B), jnp.float32),
        scratch_shapes=[pltpu.VMEM((T * EMB, B), jnp.float32),
                        pltpu.VMEM((T * 2 * U1, B), jnp.float32)],
    )(x2d, w1, b1, w2, b2, wd, bd, wc, bc)


def _pack_lstm_weights(wf_x, wf_h, bf, wb_x, wb_h, bb, u, din):
    af = jnp.concatenate([wf_x, wf_h], axis=0).T
    ab = jnp.concatenate([wb_x, wb_h], axis=0).T
    dpu = din + u
    w = jnp.zeros((8 * u, 2 * dpu), jnp.float32)
    bias = []
    for k, r0 in enumerate((0, u, 3 * u, 2 * u)):
        sc = 0.5 if k < 3 else 1.0
        w = w.at[2 * k * u:(2 * k + 1) * u, 0:dpu].set(sc * af[r0:r0 + u])
        w = w.at[(2 * k + 1) * u:(2 * k + 2) * u, dpu:2 * dpu].set(
            sc * ab[r0:r0 + u])
        bias.append(sc * bf[r0:r0 + u])
        bias.append(sc * bb[r0:r0 + u])
    b = jnp.concatenate(bias)[:, None]
    return w, b


def kernel(inputs, emb, w1f_x, w1f_h, b1f, w1b_x, w1b_h, b1b,
           w2f_x, w2f_h, b2f, w2b_x, w2b_h, b2b, Wd, bd, Wc, bc):
    v = inputs.astype(jnp.int32)
    _bm1, _qm1 = _RELAYOUT_BLK - 1, _RELAYOUT_Q - 1
    _lq = _RELAYOUT_Q.bit_length() - 1
    vp = (v & ~_bm1) + ((v & _qm1) << 2) + ((v & _bm1) >> _lq)
    idx = (vp.astype(jnp.float32).T
           .reshape(T, 4, 256).transpose(0, 2, 1).reshape(-1)
           .astype(jnp.int32))
    emb_lin = _relayout_emb(emb.T).reshape(-1, EMB)
    rows = _sc_gather(emb_lin, idx)
    x2d = rows.reshape(T * 256, 128)

    w1, b1 = _pack_lstm_weights(w1f_x, w1f_h, b1f, w1b_x, w1b_h, b1b,
                                U1, EMB)
    w2, b2 = _pack_lstm_weights(w2f_x, w2f_h, b2f, w2b_x, w2b_h, b2b,
                                U2, 2 * U1)
    out_t = _tc_forward(x2d, w1, b1, w2, b2,
                        Wd.T, bd[:, None], Wc.T, bc[:, None])
    return out_t.T


# speedup vs baseline: 1.1140x; 1.0069x over previous
import functools

import jax
import jax.numpy as jnp
from jax import lax
from jax.experimental import pallas as pl
from jax.experimental.pallas import tpu as pltpu
from jax.experimental.pallas import tpu_sc as plsc

VOCAB = 1000000
EMB = 32
T = 100
U1 = 32
U2 = 16
NCLS = 404
B = 1024


def _sc_gather(emb, idx_flat):
    info = plsc.get_sparse_core_info()
    ncores, nsub = info.num_cores, info.num_subcores
    nw = ncores * nsub
    n = idx_flat.shape[0]
    per_w = n // nw

    mesh = plsc.VectorSubcoreMesh(core_axis_name="c", subcore_axis_name="s")

    @functools.partial(
        pl.kernel,
        mesh=mesh,
        out_type=jax.ShapeDtypeStruct((n, EMB), jnp.float32),
        scratch_types=[
            pltpu.VMEM((per_w,), jnp.int32),
            pltpu.VMEM((per_w, EMB), jnp.float32),
            pltpu.SemaphoreType.DMA,
        ],
        compiler_params=pltpu.CompilerParams(use_tc_tiling_on_sc=False),
    )
    def k(table_hbm, idx_hbm, out_hbm, idx_v, rows_v, sem):
        wid = lax.axis_index("s") * ncores + lax.axis_index("c")
        base = wid * per_w
        pltpu.sync_copy(idx_hbm.at[pl.ds(base, per_w)], idx_v)
        pltpu.async_copy(table_hbm.at[idx_v], rows_v, sem).wait()
        pltpu.sync_copy(rows_v, out_hbm.at[pl.ds(base, per_w)])

    return k(emb, idx_flat)


_RELAYOUT_BLK = 32768
_RELAYOUT_NBLK = -(-VOCAB // _RELAYOUT_BLK)
_RELAYOUT_Q = _RELAYOUT_BLK // 4


def _relayout_body(embt_ref, out_ref):
    blk = embt_ref[:]
    q = _RELAYOUT_Q
    s128 = jnp.concatenate([blk[:, k * q:(k + 1) * q] for k in range(4)],
                           axis=0)
    out_ref[:] = jnp.transpose(s128)


def _relayout_emb(embt):
    return pl.pallas_call(
        _relayout_body,
        grid=(_RELAYOUT_NBLK,),
        in_specs=[pl.BlockSpec((EMB, _RELAYOUT_BLK), lambda i: (0, i))],
        out_specs=pl.BlockSpec((_RELAYOUT_BLK // 4, 128), lambda i: (i, 0)),
        out_shape=jax.ShapeDtypeStruct(
            (_RELAYOUT_NBLK * _RELAYOUT_BLK // 4, 128), jnp.float32),
    )(embt)


def _mm(a, b):
    return lax.dot_general(a, b, (((1,), (0,)), ((), ())),
                           preferred_element_type=jnp.float32)


def _tc_body(x2_ref, w1_ref, b1_ref, w2_ref, b2_ref, wd_ref, bd_ref,
             wc_ref, bc_ref, out_ref, xt_ref, x1_ref):
    def tr_step(t, _):
        blk = x2_ref[pl.ds(pl.multiple_of(t * 256, 256), 256), :]
        y = jnp.transpose(blk)
        base = pl.multiple_of(t * EMB, EMB)
        xt_ref[pl.ds(base, EMB), 0:256] = y[0:32]
        xt_ref[pl.ds(base, EMB), 256:512] = y[32:64]
        xt_ref[pl.ds(base, EMB), 512:768] = y[64:96]
        xt_ref[pl.ds(base, EMB), 768:1024] = y[96:128]
        return 0

    lax.fori_loop(0, T, tr_step, 0, unroll=4)

    w1 = w1_ref[:]
    b1 = b1_ref[:]

    def l1_step(t, carry):
        h, c = carry
        xf = xt_ref[pl.ds(pl.multiple_of(t * EMB, EMB), EMB), :]
        xb = xt_ref[pl.ds(pl.multiple_of((T - 1 - t) * EMB, EMB), EMB), :]
        s = jnp.concatenate([xf, h[0:U1], xb, h[U1:2 * U1]], axis=0)
        z = _mm(w1, s) + b1
        zs = jnp.tanh(z[0:6 * U1]) * 0.5 + 0.5
        g = jnp.tanh(z[6 * U1:8 * U1])
        i = zs[0:2 * U1]
        f = zs[2 * U1:4 * U1]
        o = zs[4 * U1:6 * U1]
        c2 = f * c + i * g
        h2 = o * jnp.tanh(c2)
        x1_ref[pl.ds(pl.multiple_of(t * 2 * U1, 2 * U1), U1), :] = h2[0:U1]
        x1_ref[pl.ds(pl.multiple_of((T - 1 - t) * 2 * U1 + U1, U1), U1), :] = \
            h2[U1:2 * U1]
        return h2, c2

    zero1 = jnp.zeros((2 * U1, B), jnp.float32)
    lax.fori_loop(0, T, l1_step, (zero1, zero1), unroll=10)

    w2 = w2_ref[:]
    b2 = b2_ref[:]

    def l2_step(t, carry):
        h, c = carry
        x1f = x1_ref[pl.ds(pl.multiple_of(t * 2 * U1, 2 * U1), 2 * U1), :]
        x1b = x1_ref[pl.ds(pl.multiple_of((T - 1 - t) * 2 * U1, 2 * U1),
                           2 * U1), :]
        s = jnp.concatenate([x1f, h[0:U2], x1b, h[U2:2 * U2]], axis=0)
        z = _mm(w2, s) + b2
        zs = jnp.tanh(z[0:6 * U2]) * 0.5 + 0.5
        g = jnp.tanh(z[6 * U2:8 * U2])
        i = zs[0:2 * U2]
        f = zs[2 * U2:4 * U2]
        o = zs[4 * U2:6 * U2]
        c2 = f * c + i * g
        h2 = o * jnp.tanh(c2)
        return h2, c2

    zero2 = jnp.zeros((2 * U2, B), jnp.float32)
    h2, _ = lax.fori_loop(0, T, l2_step, (zero2, zero2), unroll=10)

    d = jnp.maximum(_mm(wd_ref[:], h2) + bd_ref[:], 0.0)
    logits = _mm(wc_ref[:], d) + bc_ref[:]
    m = jnp.max(logits, axis=0, keepdims=True)
    e = jnp.exp(logits - m)
    out_ref[:] = e / jnp.sum(e, axis=0, keepdims=True)


def _tc_forward(x2d, w1, b1, w2, b2, wd, bd, wc, bc):
    return pl.pallas_call(
        _tc_body,
        out_shape=jax.ShapeDtypeStruct((NCLS, B), jnp.float32),
        scratch_shapes=[pltpu.VMEM((T * EMB, B), jnp.float32),
                        pltpu.VMEM((T * 2 * U1, B), jnp.float32)],
    )(x2d, w1, b1, w2, b2, wd, bd, wc, bc)


def _pack_lstm_weights(wf_x, wf_h, bf, wb_x, wb_h, bb, u, din):
    af = jnp.concatenate([wf_x, wf_h], axis=0).T
    ab = jnp.concatenate([wb_x, wb_h], axis=0).T
    dpu = din + u
    w = jnp.zeros((8 * u, 2 * dpu), jnp.float32)
    bias = []
    for k, r0 in enumerate((0, u, 3 * u, 2 * u)):
        sc = 0.5 if k < 3 else 1.0
        w = w.at[2 * k * u:(2 * k + 1) * u, 0:dpu].set(sc * af[r0:r0 + u])
        w = w.at[(2 * k + 1) * u:(2 * k + 2) * u, dpu:2 * dpu].set(
            sc * ab[r0:r0 + u])
        bias.append(sc * bf[r0:r0 + u])
        bias.append(sc * bb[r0:r0 + u])
    b = jnp.concatenate(bias)[:, None]
    return w, b


def kernel(inputs, emb, w1f_x, w1f_h, b1f, w1b_x, w1b_h, b1b,
           w2f_x, w2f_h, b2f, w2b_x, w2b_h, b2b, Wd, bd, Wc, bc):
    v = inputs.astype(jnp.int32)
    _bm1, _qm1 = _RELAYOUT_BLK - 1, _RELAYOUT_Q - 1
    _lq = _RELAYOUT_Q.bit_length() - 1
    vp = (v & ~_bm1) + ((v & _qm1) << 2) + ((v & _bm1) >> _lq)
    idx = (vp.astype(jnp.float32).T
           .reshape(T, 4, 256).transpose(0, 2, 1).reshape(-1)
           .astype(jnp.int32))
    emb_lin = _relayout_emb(emb.T).reshape(-1, EMB)
    rows = _sc_gather(emb_lin, idx)
    x2d = rows.reshape(T * 256, 128)

    w1, b1 = _pack_lstm_weights(w1f_x, w1f_h, b1f, w1b_x, w1b_h, b1b,
                                U1, EMB)
    w2, b2 = _pack_lstm_weights(w2f_x, w2f_h, b2f, w2b_x, w2b_h, b2b,
                                U2, 2 * U1)
    out_t = _tc_forward(x2d, w1, b1, w2, b2,
                        Wd.T, bd[:, None], Wc.T, bc[:, None])
    return out_t.T
